# baseline (speedup 1.0000x reference)
import jax
import jax.numpy as jnp
from jax.experimental import pallas as pl
from jax.experimental.pallas import tpu as pltpu


def _grab_units(x, chans, rows, cols, rows_v, cols_v):
    B, C, H, W = x.shape
    U = chans.shape[0]
    CW = 128
    CH = 8

    def body(chans_ref, rows_ref, cols_ref, rowsv_ref, colsv_ref, x_ref,
             out_ref, buf, sem):
        for u in range(U):
            c = chans_ref[u]
            r_al = pl.multiple_of((rows_ref[u] // CH) * CH, CH)
            w_al = pl.multiple_of((cols_ref[u] // CW) * CW, CW)
            pltpu.make_async_copy(
                x_ref.at[:, c, pl.ds(r_al, CH), pl.ds(w_al, CW)],
                buf.at[:, u],
                sem.at[u % 8],
            ).start()
        for u in range(U):
            pltpu.make_async_copy(
                x_ref.at[:, 0, pl.ds(0, CH), pl.ds(0, CW)],
                buf.at[:, u],
                sem.at[u % 8],
            ).wait()
        rv = rowsv_ref[...]
        wv = colsv_ref[...]
        rm = rv % CH
        wm = wv % CW
        j_idx = jax.lax.broadcasted_iota(jnp.int32, (U, CH, CW), 1)
        l_idx = jax.lax.broadcasted_iota(jnp.int32, (U, CH, CW), 2)
        mask = (j_idx == rm[:, None, None]) & (l_idx == wm[:, None, None])
        masked = jnp.where(mask[None, :, :, :], buf[...], 0.0)
        out_ref[...] = jnp.sum(masked, axis=(2, 3))

    return pl.pallas_call(
        body,
        in_specs=[
            pl.BlockSpec(memory_space=pltpu.MemorySpace.SMEM),
            pl.BlockSpec(memory_space=pltpu.MemorySpace.SMEM),
            pl.BlockSpec(memory_space=pltpu.MemorySpace.SMEM),
            pl.BlockSpec(memory_space=pltpu.MemorySpace.VMEM),
            pl.BlockSpec(memory_space=pltpu.MemorySpace.VMEM),
            pl.BlockSpec(memory_space=pltpu.MemorySpace.HBM),
        ],
        out_specs=pl.BlockSpec(memory_space=pltpu.MemorySpace.VMEM),
        out_shape=jax.ShapeDtypeStruct((B, U), jnp.float32),
        scratch_shapes=[
            pltpu.VMEM((B, U, CH, CW), jnp.float32),
            pltpu.SemaphoreType.DMA((8,)),
        ],
    )(chans, rows, cols, rows_v, cols_v, x)


def kernel(x, chans, coords):
    ch = chans.astype(jnp.int32)
    r = coords[:, 0].astype(jnp.int32)
    c = coords[:, 1].astype(jnp.int32)
    return _grab_units(x, ch, r, c, r, c)
